# baseline (speedup 1.0000x reference)
import functools

import jax
import jax.numpy as jnp
from jax import lax
from jax.experimental import pallas as pl
from jax.experimental.pallas import tpu as pltpu
from jax.experimental.pallas import tpu_sc as plsc

BATCH = 16384
HIST = 200
EMBED = 2
N_ROWS = 1_000_000
N_TOTAL = BATCH * HIST
NC, NS = 2, 16
NW = NC * NS
PER_W = N_TOTAL // NW
CHUNK = 12800
STEPS = PER_W // CHUNK
LANES = 16

_mesh = plsc.VectorSubcoreMesh(core_axis_name="c", subcore_axis_name="s")


@functools.partial(
    pl.kernel,
    out_type=jax.ShapeDtypeStruct((N_TOTAL * EMBED,), jnp.float32),
    mesh=_mesh,
    scratch_types=[
        pltpu.VMEM((CHUNK,), jnp.int32),
        pltpu.VMEM((CHUNK * EMBED,), jnp.int32),
        pltpu.VMEM((CHUNK * EMBED,), jnp.float32),
        pltpu.SemaphoreType.DMA,
    ],
    compiler_params=pltpu.CompilerParams(
        use_tc_tiling_on_sc=False, needs_layout_passes=False),
)
def _gather_kernel(idx2_hbm, table_hbm, out_hbm, idx_v, didx_v, vals_v, sem):
    wid = lax.axis_index("s") * NC + lax.axis_index("c")
    base = wid * PER_W
    pos_e = lax.iota(jnp.int32, LANES) * 2
    pos_o = pos_e + 1

    def body(g, carry):
        off = base + g * CHUNK
        pltpu.sync_copy(idx2_hbm.at[pl.ds(off, CHUNK)], idx_v)

        def expand(k, c):
            v = idx_v[pl.ds(k * LANES, LANES)]
            p = pos_e + k * (2 * LANES)
            plsc.store_scatter(didx_v, [p], v)
            plsc.store_scatter(didx_v, [p + 1], v + 1)
            return c

        lax.fori_loop(0, CHUNK // LANES, expand, 0)

        pltpu.async_copy(table_hbm.at[didx_v], vals_v, sem).wait()
        pltpu.sync_copy(vals_v, out_hbm.at[pl.ds(off * EMBED, CHUNK * EMBED)])
        return carry

    lax.fori_loop(0, STEPS, body, 0)


def kernel(x, W):
    idx2 = x.reshape(N_TOTAL).astype(jnp.int32) * 2
    out = _gather_kernel(idx2, W.reshape(N_ROWS * EMBED))
    return out.reshape(BATCH, HIST, EMBED)
